# baseline (speedup 1.0000x reference)
import jax
import jax.numpy as jnp
from jax import lax
from jax.experimental import pallas as pl

NUM_ACTIONS = 6
ACTION_DIM = 32
BATCH = 16384
HIST = 200

R = 512
GRP = 4
NG = HIST // GRP
W = GRP * ACTION_DIM


def _tc_body(act_ref, ttab_ref, out_ref):
    act = act_ref[0, 0]
    lane = lax.broadcasted_iota(jnp.int32, (R, W), 1)
    g = lane // ACTION_DIM
    idx = act[:, 0:1]
    for j in range(1, GRP):
        idx = jnp.where(g == j, act[:, j:j + 1], idx)
    acc = jnp.broadcast_to(ttab_ref[0:1, :], (R, W))
    for a in range(1, NUM_ACTIONS):
        acc = jnp.where(idx == a, ttab_ref[a:a + 1, :], acc)
    out_ref[...] = acc


def kernel(action, table):
    ttab = jnp.tile(table, (1, GRP))
    act4 = action.reshape(BATCH // R, R, NG, GRP).transpose(0, 2, 1, 3)

    out = pl.pallas_call(
        _tc_body,
        grid=(BATCH // R, NG),
        in_specs=[
            pl.BlockSpec((1, 1, R, GRP), lambda i, g: (i, g, 0, 0)),
            pl.BlockSpec((NUM_ACTIONS, W), lambda i, g: (0, 0)),
        ],
        out_specs=pl.BlockSpec((R, W), lambda i, g: (i, g)),
        out_shape=jax.ShapeDtypeStruct((BATCH, HIST * ACTION_DIM), jnp.float32),
    )(act4, ttab)
    return out


# speedup vs baseline: 2.3844x; 2.3844x over previous
import jax
import jax.numpy as jnp
from jax import lax
from jax.experimental import pallas as pl
from jax.experimental.pallas import tpu as pltpu
from jax.experimental.pallas import tpu_sc as plsc

NUM_ACTIONS = 6
ACTION_DIM = 32
BATCH = 16384
HIST = 200

NC = 2
NS = 16
NW = NC * NS
L = 16

TUP = 4
ROW_T = HIST // TUP
G = 8
CHUNK_A = G * HIST
CHUNK_T = G * ROW_T
ROW_W = TUP * ACTION_DIM
NCHUNKS = BATCH // G
CPW = NCHUNKS // NW
TVECS = CHUNK_T // L


_GPIECES = [(i * 128, 128) for i in range(CHUNK_T // 128)]
if CHUNK_T % 128:
    _GPIECES.append((CHUNK_T - CHUNK_T % 128, CHUNK_T % 128))


def _sc_body(act_hbm, ptab_hbm, out_hbm, a0_v, a1_v, t0_v, t1_v, r0_v, r1_v,
             is0, is1, gs0, gs1, ws0, ws1):
    wid = lax.axis_index("s") * NC + lax.axis_index("c")
    base = wid * CPW
    i16 = lax.iota(jnp.int32, 16)
    acts, tidx, rows = (a0_v, a1_v), (t0_v, t1_v), (r0_v, r1_v)
    isem, gsem, wsem = (is0, is1), (gs0, gs1), (ws0, ws1)

    def fire_idx(i, b):
        pltpu.async_copy(act_hbm.at[base + i], acts[b], isem[b])

    def drain_idx(b):
        pltpu.make_async_copy(act_hbm.at[0], acts[b], isem[b]).wait()

    def fire_gathers(b):
        for t in range(TVECS):
            pos = i16 * TUP + t * (L * TUP)
            a0 = plsc.load_gather(acts[b], [pos])
            a1 = plsc.load_gather(acts[b], [pos + 1])
            a2 = plsc.load_gather(acts[b], [pos + 2])
            a3 = plsc.load_gather(acts[b], [pos + 3])
            idx = ((a0 * NUM_ACTIONS + a1) * NUM_ACTIONS + a2) * NUM_ACTIONS + a3
            tidx[b][pl.ds(t * L, L)] = idx
        for off, ln in _GPIECES:
            pltpu.async_copy(
                ptab_hbm.at[tidx[b].at[pl.ds(off, ln)]],
                rows[b].at[pl.ds(off, ln)],
                gsem[b],
            )

    def drain_gathers(b):
        pltpu.make_async_copy(out_hbm.at[0], rows[b], gsem[b]).wait()

    def fire_write(i, b):
        pltpu.async_copy(rows[b], out_hbm.at[base + i], wsem[b])

    def drain_write(b):
        pltpu.make_async_copy(out_hbm.at[0], rows[b], wsem[b]).wait()

    def slot(i, b, first, last):
        @pl.when(jnp.logical_not(first))
        def _():
            drain_write(b)
        drain_idx(b)
        fire_gathers(b)
        @pl.when(jnp.logical_not(last))
        def _():
            fire_idx(i + 2, b)
        @pl.when(i > 0)
        def _():
            drain_gathers(1 - b)
            fire_write(i - 1, 1 - b)

    fire_idx(0, 0)
    fire_idx(1, 1)

    @pl.loop(0, CPW, step=2)
    def _pair(c0):
        slot(c0, 0, c0 == 0, c0 + 2 >= CPW)
        slot(c0 + 1, 1, c0 == 0, c0 + 3 >= CPW)

    drain_gathers((CPW - 1) % 2)
    fire_write(CPW - 1, (CPW - 1) % 2)
    drain_write(0)
    drain_write(1)


def kernel(action, table):
    aidx = jnp.arange(NUM_ACTIONS**TUP, dtype=jnp.int32)
    parts = []
    for k in range(TUP):
        ak = (aidx // (NUM_ACTIONS ** (TUP - 1 - k))) % NUM_ACTIONS
        parts.append(jnp.take(table, ak, axis=0))
    ptab = jnp.concatenate(parts, axis=1)

    act2 = action.reshape(NCHUNKS, CHUNK_A)
    kfn = pl.kernel(
        _sc_body,
        out_type=jax.ShapeDtypeStruct((NCHUNKS, CHUNK_T, ROW_W), jnp.float32),
        mesh=plsc.VectorSubcoreMesh(core_axis_name="c", subcore_axis_name="s"),
        compiler_params=pltpu.CompilerParams(needs_layout_passes=False),
        scratch_types=[
            pltpu.VMEM((CHUNK_A,), jnp.int32),
            pltpu.VMEM((CHUNK_A,), jnp.int32),
            pltpu.VMEM((CHUNK_T,), jnp.int32),
            pltpu.VMEM((CHUNK_T,), jnp.int32),
            pltpu.VMEM((CHUNK_T, ROW_W), jnp.float32),
            pltpu.VMEM((CHUNK_T, ROW_W), jnp.float32),
            pltpu.SemaphoreType.DMA,
            pltpu.SemaphoreType.DMA,
            pltpu.SemaphoreType.DMA,
            pltpu.SemaphoreType.DMA,
            pltpu.SemaphoreType.DMA,
            pltpu.SemaphoreType.DMA,
        ],
    )
    out3 = kfn(act2, ptab)
    return out3.reshape(BATCH, HIST * ACTION_DIM)


# speedup vs baseline: 3.3473x; 1.4038x over previous
import jax
import jax.numpy as jnp
from jax import lax
from jax.experimental import pallas as pl
from jax.experimental.pallas import tpu as pltpu
from jax.experimental.pallas import tpu_sc as plsc

NUM_ACTIONS = 6
ACTION_DIM = 32
BATCH = 16384
HIST = 200

NC = 2
NS = 16
NW = NC * NS
L = 16

TUP = 4
ROW_T = HIST // TUP
G = 8
CHUNK_A = G * HIST
CHUNK_T = G * ROW_T
ROW_W = TUP * ACTION_DIM
NCHUNKS = BATCH // G
CPW = NCHUNKS // NW
TVECS = CHUNK_T // L


_GPIECES = [(i * 128, 128) for i in range(CHUNK_T // 128)]
if CHUNK_T % 128:
    _GPIECES.append((CHUNK_T - CHUNK_T % 128, CHUNK_T % 128))


def _sc_body(act_hbm, ptab_hbm, out_hbm, lut_s, a0_v, a1_v, t0_v, t1_v,
             r0_v, r1_v, is0, is1, gs0, gs1, ws0, ws1):
    wid = lax.axis_index("s") * NC + lax.axis_index("c")
    base = wid * CPW
    i16 = lax.iota(jnp.int32, 16)
    acts, tidx, rows = (a0_v, a1_v), (t0_v, t1_v), (r0_v, r1_v)
    isem, gsem, wsem = (is0, is1), (gs0, gs1), (ws0, ws1)

    @pl.when(lax.axis_index("s") == 0)
    def _():
        pltpu.sync_copy(ptab_hbm, lut_s)

    plsc.subcore_barrier()

    def fire_idx(i, b):
        pltpu.async_copy(act_hbm.at[base + i], acts[b], isem[b])

    def drain_idx(b):
        pltpu.make_async_copy(act_hbm.at[0], acts[b], isem[b]).wait()

    def fire_gathers(b):
        for t in range(TVECS):
            pos = i16 * TUP + t * (L * TUP)
            a0 = plsc.load_gather(acts[b], [pos])
            a1 = plsc.load_gather(acts[b], [pos + 1])
            a2 = plsc.load_gather(acts[b], [pos + 2])
            a3 = plsc.load_gather(acts[b], [pos + 3])
            idx = ((a0 * NUM_ACTIONS + a1) * NUM_ACTIONS + a2) * NUM_ACTIONS + a3
            tidx[b][pl.ds(t * L, L)] = idx
        for off, ln in _GPIECES:
            pltpu.async_copy(
                lut_s.at[tidx[b].at[pl.ds(off, ln)]],
                rows[b].at[pl.ds(off, ln)],
                gsem[b],
            )

    def drain_gathers(b):
        pltpu.make_async_copy(out_hbm.at[0], rows[b], gsem[b]).wait()

    def fire_write(i, b):
        pltpu.async_copy(rows[b], out_hbm.at[base + i], wsem[b])

    def drain_write(b):
        pltpu.make_async_copy(out_hbm.at[0], rows[b], wsem[b]).wait()

    def slot(i, b, first, last):
        @pl.when(jnp.logical_not(first))
        def _():
            drain_write(b)
        drain_idx(b)
        fire_gathers(b)
        @pl.when(jnp.logical_not(last))
        def _():
            fire_idx(i + 2, b)
        @pl.when(i > 0)
        def _():
            drain_gathers(1 - b)
            fire_write(i - 1, 1 - b)

    fire_idx(0, 0)
    fire_idx(1, 1)

    @pl.loop(0, CPW, step=2)
    def _pair(c0):
        slot(c0, 0, c0 == 0, c0 + 2 >= CPW)
        slot(c0 + 1, 1, c0 == 0, c0 + 3 >= CPW)

    drain_gathers((CPW - 1) % 2)
    fire_write(CPW - 1, (CPW - 1) % 2)
    drain_write(0)
    drain_write(1)


def kernel(action, table):
    aidx = jnp.arange(NUM_ACTIONS**TUP, dtype=jnp.int32)
    parts = []
    for k in range(TUP):
        ak = (aidx // (NUM_ACTIONS ** (TUP - 1 - k))) % NUM_ACTIONS
        parts.append(jnp.take(table, ak, axis=0))
    ptab = jnp.concatenate(parts, axis=1)

    act2 = action.reshape(NCHUNKS, CHUNK_A)
    kfn = pl.kernel(
        _sc_body,
        out_type=jax.ShapeDtypeStruct((NCHUNKS, CHUNK_T, ROW_W), jnp.float32),
        mesh=plsc.VectorSubcoreMesh(core_axis_name="c", subcore_axis_name="s"),
        compiler_params=pltpu.CompilerParams(needs_layout_passes=False),
        scratch_types=[
            pltpu.VMEM_SHARED((NUM_ACTIONS**TUP, ROW_W), jnp.float32),
            pltpu.VMEM((CHUNK_A,), jnp.int32),
            pltpu.VMEM((CHUNK_A,), jnp.int32),
            pltpu.VMEM((CHUNK_T,), jnp.int32),
            pltpu.VMEM((CHUNK_T,), jnp.int32),
            pltpu.VMEM((CHUNK_T, ROW_W), jnp.float32),
            pltpu.VMEM((CHUNK_T, ROW_W), jnp.float32),
            pltpu.SemaphoreType.DMA,
            pltpu.SemaphoreType.DMA,
            pltpu.SemaphoreType.DMA,
            pltpu.SemaphoreType.DMA,
            pltpu.SemaphoreType.DMA,
            pltpu.SemaphoreType.DMA,
        ],
    )
    out3 = kfn(act2, ptab)
    return out3.reshape(BATCH, HIST * ACTION_DIM)


# speedup vs baseline: 7.3374x; 2.1920x over previous
import jax
import jax.numpy as jnp
from jax import lax
from jax.experimental import pallas as pl
from jax.experimental.pallas import tpu as pltpu
from jax.experimental.pallas import tpu_sc as plsc

NUM_ACTIONS = 6
ACTION_DIM = 32
BATCH = 16384
HIST = 200

NC = 2
NS = 16
NW = NC * NS
L = 16

TUP = 4
ROW_T = HIST // TUP
G = 8
CHUNK_A = G * HIST
CHUNK_T = G * ROW_T
ROW_W = TUP * ACTION_DIM
OUT_W = HIST * ACTION_DIM
NCHUNKS = BATCH // G
CPW = NCHUNKS // NW
TVECS = CHUNK_T // L


def _sc_body(act_hbm, ptab_hbm, out_hbm, lut_s, a0_v, a1_v, t0_v, t1_v,
             r0_v, r1_v, is0, is1, gs0, gs1, ws0, ws1):
    wid = lax.axis_index("s") * NC + lax.axis_index("c")
    base = wid * CPW
    i16 = lax.iota(jnp.int32, 16)
    perm16 = (i16 % G) * HIST + (i16 // G) * TUP
    acts, tidx, rows = (a0_v, a1_v), (t0_v, t1_v), (r0_v, r1_v)
    isem, gsem, wsem = (is0, is1), (gs0, gs1), (ws0, ws1)

    @pl.when(lax.axis_index("s") == 0)
    def _():
        pltpu.sync_copy(ptab_hbm, lut_s)

    plsc.subcore_barrier()

    def fire_idx(i, b):
        pltpu.async_copy(act_hbm.at[base + i], acts[b], isem[b])

    def drain_idx(b):
        pltpu.make_async_copy(act_hbm.at[0], acts[b], isem[b]).wait()

    def fire_gathers(b):
        for j in range(TVECS):
            pos = perm16 + j * (2 * TUP)
            a0 = plsc.load_gather(acts[b], [pos])
            a1 = plsc.load_gather(acts[b], [pos + 1])
            a2 = plsc.load_gather(acts[b], [pos + 2])
            a3 = plsc.load_gather(acts[b], [pos + 3])
            idx = ((a0 * NUM_ACTIONS + a1) * NUM_ACTIONS + a2) * NUM_ACTIONS + a3
            tidx[b][pl.ds(j * L, L)] = idx
        for t in range(ROW_T):
            pltpu.async_copy(
                lut_s.at[tidx[b].at[pl.ds(t * G, G)]],
                rows[b].at[:, pl.ds(t * ROW_W, ROW_W)],
                gsem[b],
            )

    def drain_gathers(b):
        pltpu.make_async_copy(out_hbm.at[pl.ds(0, G)], rows[b], gsem[b]).wait()

    def fire_write(i, b):
        pltpu.async_copy(rows[b], out_hbm.at[pl.ds((base + i) * G, G)], wsem[b])

    def drain_write(b):
        pltpu.make_async_copy(out_hbm.at[pl.ds(0, G)], rows[b], wsem[b]).wait()

    def slot(i, b, first, last):
        @pl.when(jnp.logical_not(first))
        def _():
            drain_write(b)
        drain_idx(b)
        fire_gathers(b)
        @pl.when(jnp.logical_not(last))
        def _():
            fire_idx(i + 2, b)
        @pl.when(i > 0)
        def _():
            drain_gathers(1 - b)
            fire_write(i - 1, 1 - b)

    fire_idx(0, 0)
    fire_idx(1, 1)

    @pl.loop(0, CPW, step=2)
    def _pair(c0):
        slot(c0, 0, c0 == 0, c0 + 2 >= CPW)
        slot(c0 + 1, 1, c0 == 0, c0 + 3 >= CPW)

    drain_gathers((CPW - 1) % 2)
    fire_write(CPW - 1, (CPW - 1) % 2)
    drain_write(0)
    drain_write(1)


def kernel(action, table):
    aidx = jnp.arange(NUM_ACTIONS**TUP, dtype=jnp.int32)
    parts = []
    for k in range(TUP):
        ak = (aidx // (NUM_ACTIONS ** (TUP - 1 - k))) % NUM_ACTIONS
        parts.append(jnp.take(table, ak, axis=0))
    ptab = jnp.concatenate(parts, axis=1)

    act2 = action.reshape(NCHUNKS, CHUNK_A)
    kfn = pl.kernel(
        _sc_body,
        out_type=jax.ShapeDtypeStruct((BATCH, OUT_W), jnp.float32),
        mesh=plsc.VectorSubcoreMesh(core_axis_name="c", subcore_axis_name="s"),
        compiler_params=pltpu.CompilerParams(needs_layout_passes=False),
        scratch_types=[
            pltpu.VMEM_SHARED((NUM_ACTIONS**TUP, ROW_W), jnp.float32),
            pltpu.VMEM((CHUNK_A,), jnp.int32),
            pltpu.VMEM((CHUNK_A,), jnp.int32),
            pltpu.VMEM((CHUNK_T,), jnp.int32),
            pltpu.VMEM((CHUNK_T,), jnp.int32),
            pltpu.VMEM((G, OUT_W), jnp.float32),
            pltpu.VMEM((G, OUT_W), jnp.float32),
            pltpu.SemaphoreType.DMA,
            pltpu.SemaphoreType.DMA,
            pltpu.SemaphoreType.DMA,
            pltpu.SemaphoreType.DMA,
            pltpu.SemaphoreType.DMA,
            pltpu.SemaphoreType.DMA,
        ],
    )
    return kfn(act2, ptab)
